# baseline (speedup 1.0000x reference)
import functools

import jax
import jax.numpy as jnp
from jax import lax
from jax.experimental import pallas as pl
from jax.experimental.pallas import tpu as pltpu
from jax.experimental.pallas import tpu_sc as plsc

_IDX_CHUNK = 128
_TBLK = 16384
_SPLIT = 44
_QCH = 256


def _pairize_body(xT_ref, out_ref):
    x = xT_ref[...]
    h = x.shape[1] // 2
    out_ref[...] = jnp.concatenate([x[:, :h].T, x[:, h:].T], axis=1)


@functools.cache
def _pairize_tc_fn(A, V, S):
    last = (V - 1) // _TBLK
    return pl.pallas_call(
        _pairize_body,
        grid=(S + 1,),
        in_specs=[
            pl.BlockSpec((A, _TBLK), lambda i: (0, jnp.where(i < S, i, last)))
        ],
        out_specs=pl.BlockSpec((_TBLK // 2, 2 * A), lambda i: (i, 0)),
        out_shape=jax.ShapeDtypeStruct(((S + 1) * (_TBLK // 2), 2 * A), jnp.float32),
    )


@functools.cache
def _pairize_sc_fn(A, V, S):
    info = plsc.get_sparse_core_info()
    last = (V - 1) // _TBLK
    nb = last - S
    half = _TBLK // 2
    mesh = plsc.VectorSubcoreMesh(core_axis_name="c", subcore_axis_name="s")

    @functools.partial(
        pl.kernel,
        out_type=jax.ShapeDtypeStruct((nb * half, 2 * A), jnp.float32),
        mesh=mesh,
        compiler_params=pltpu.CompilerParams(needs_layout_passes=False),
        scratch_types=[
            pltpu.VMEM((2, A, _QCH), jnp.float32),
            pltpu.VMEM((_QCH, 2 * A), jnp.float32),
        ],
    )
    def pairize(tT_hbm, out_hbm, slab_v, outb_v):
        wid = lax.axis_index("s") * info.num_cores + lax.axis_index("c")
        lanes16 = lax.iota(jnp.int32, 16)
        for u in range(nb):
            c = S + u
            lane0 = c * _TBLK + wid * _QCH
            pltpu.sync_copy(tT_hbm.at[:, pl.ds(lane0, _QCH)], slab_v.at[0])
            pltpu.sync_copy(
                tT_hbm.at[:, pl.ds(lane0 + half, _QCH)], slab_v.at[1]
            )

            def qstep(q, carry):
                for v in range(2 * A // 16):
                    hh = v // (A // 16)
                    a0 = 16 * (v % (A // 16))
                    vals = plsc.load_gather(
                        slab_v,
                        [
                            jnp.full((16,), hh, jnp.int32),
                            a0 + lanes16,
                            jnp.full((16,), q, jnp.int32),
                        ],
                    )
                    outb_v[q, pl.ds(16 * v, 16)] = vals
                return carry

            lax.fori_loop(0, _QCH, qstep, 0)
            pltpu.sync_copy(
                outb_v, out_hbm.at[pl.ds(u * half + wid * _QCH, _QCH)]
            )

    return pairize


@functools.cache
def _gather_fn(B, Vp, W):
    info = plsc.get_sparse_core_info()
    nw = info.num_cores * info.num_subcores
    b_per_w = B // nw
    n_ch = b_per_w // _IDX_CHUNK
    mesh = plsc.VectorSubcoreMesh(core_axis_name="c", subcore_axis_name="s")

    @functools.partial(
        pl.kernel,
        out_type=jax.ShapeDtypeStruct((B, W), jnp.float32),
        mesh=mesh,
        scratch_types=[
            pltpu.VMEM((n_ch, _IDX_CHUNK), jnp.int32),
            pltpu.VMEM((b_per_w, W), jnp.float32),
            pltpu.SemaphoreType.DMA,
        ],
    )
    def gather(idx_hbm, table_hbm, out_hbm, idx_v, rows_v, sem):
        wid = lax.axis_index("s") * info.num_cores + lax.axis_index("c")
        base = wid * b_per_w
        pltpu.sync_copy(idx_hbm.at[wid], idx_v)
        copies = [
            pltpu.async_copy(
                table_hbm.at[idx_v.at[j]],
                rows_v.at[pl.ds(j * _IDX_CHUNK, _IDX_CHUNK)],
                sem,
            )
            for j in range(n_ch)
        ]
        for c in copies:
            c.wait()
        pltpu.sync_copy(rows_v, out_hbm.at[pl.ds(base, b_per_w)])

    return gather


def _sample_body(ptc_ref, psc_ref, gtc_ref, gsc_ref, act_ref, probs_ref):
    p = jnp.maximum(
        jnp.concatenate([ptc_ref[...], psc_ref[...]], axis=1), 1e-30
    )
    gg = jnp.concatenate([gtc_ref[...], gsc_ref[...]], axis=1)
    A = p.shape[1] // 4
    z = jnp.log(p) + gg
    m = jnp.max(z, axis=1, keepdims=True)
    ii = lax.broadcasted_iota(jnp.int32, z.shape, 1)
    act_ref[...] = jnp.min(jnp.where(z == m, ii, z.shape[1]), axis=1) & (A - 1)
    q = [p[:, i * A : (i + 1) * A] for i in range(4)]
    gq = [gg[:, i * A : (i + 1) * A] for i in range(4)]
    s1 = jnp.where(gq[0] > gq[1], q[0], q[1])
    g1 = jnp.maximum(gq[0], gq[1])
    s2 = jnp.where(gq[2] > gq[3], q[2], q[3])
    g2 = jnp.maximum(gq[2], gq[3])
    probs_ref[...] = jnp.where(g1 > g2, s1, s2)


@functools.cache
def _sample_fn(B, W, blk):
    A = W // 2
    return pl.pallas_call(
        _sample_body,
        grid=(B // blk,),
        in_specs=[pl.BlockSpec((blk, W), lambda i: (i, 0))] * 4,
        out_specs=[
            pl.BlockSpec((blk,), lambda i: (i,)),
            pl.BlockSpec((blk, A), lambda i: (i, 0)),
        ],
        out_shape=[
            jax.ShapeDtypeStruct((B,), jnp.int32),
            jax.ShapeDtypeStruct((B, A), jnp.float32),
        ],
    )


def kernel(state, model):
    B = state.shape[0]
    V, A = model.shape
    info = plsc.get_sparse_core_info()
    nw = info.num_cores * info.num_subcores
    b_per_w = B // nw
    S = _SPLIT
    half = _TBLK // 2
    last = (V - 1) // _TBLK
    c = state // _TBLK
    q = state & (half - 1)
    hb = ((state // half) & 1).astype(jnp.bool_)
    is_tc = (c < S) | (c == last)
    pos = jnp.arange(B, dtype=jnp.int32)
    k_tc = jnp.where(
        is_tc,
        jnp.where(c == last, S * half, c * half) + q,
        pos % ((S + 1) * half),
    )
    k_sc = jnp.where(is_tc, pos % ((last - S) * half), (c - S) * half + q)
    g = jax.random.gumbel(jax.random.key(42), (B, A), jnp.float32)
    ninf = jnp.float32(-jnp.inf)
    h2 = hb[:, None]
    t2 = is_tc[:, None]
    gg_tc = jnp.concatenate(
        [jnp.where(t2 & ~h2, g, ninf), jnp.where(t2 & h2, g, ninf)], axis=1
    )
    gg_sc = jnp.concatenate(
        [jnp.where(~t2 & ~h2, g, ninf), jnp.where(~t2 & h2, g, ninf)], axis=1
    )
    table_tc = _pairize_tc_fn(A, V, S)(model.T)
    table_sc = _pairize_sc_fn(A, V, S)(model.T)
    idx_tc = k_tc.reshape(nw, b_per_w // _IDX_CHUNK, _IDX_CHUNK)
    idx_sc = k_sc.reshape(nw, b_per_w // _IDX_CHUNK, _IDX_CHUNK)
    pairs_tc = _gather_fn(B, table_tc.shape[0], 2 * A)(idx_tc, table_tc)
    pairs_sc = _gather_fn(B, table_sc.shape[0], 2 * A)(idx_sc, table_sc)
    actions, action_probs = _sample_fn(B, 2 * A, 2048)(
        pairs_tc, pairs_sc, gg_tc, gg_sc
    )
    return actions, action_probs


# speedup vs baseline: 2.1077x; 2.1077x over previous
import functools

import jax
import jax.numpy as jnp
from jax import lax
from jax.experimental import pallas as pl
from jax.experimental.pallas import tpu as pltpu
from jax.experimental.pallas import tpu_sc as plsc

_IDX_CHUNK = 128
_TBLK = 32768


def _pairize_body(xT_ref, out_ref):
    x = xT_ref[...]
    h = x.shape[1] // 2
    out_ref[...] = jnp.concatenate([x[:, :h].T, x[:, h:].T], axis=1)


@functools.cache
def _pairize_fn(A, V):
    nblk = (V + _TBLK - 1) // _TBLK
    return pl.pallas_call(
        _pairize_body,
        grid=(nblk,),
        in_specs=[pl.BlockSpec((A, _TBLK), lambda i: (0, i))],
        out_specs=pl.BlockSpec((_TBLK // 2, 2 * A), lambda i: (i, 0)),
        out_shape=jax.ShapeDtypeStruct((nblk * (_TBLK // 2), 2 * A), jnp.float32),
    )


@functools.cache
def _gather_fn(B, Vp, W):
    info = plsc.get_sparse_core_info()
    nw = info.num_cores * info.num_subcores
    b_per_w = B // nw
    n_ch = b_per_w // _IDX_CHUNK
    mesh = plsc.VectorSubcoreMesh(core_axis_name="c", subcore_axis_name="s")

    @functools.partial(
        pl.kernel,
        out_type=jax.ShapeDtypeStruct((B, W), jnp.float32),
        mesh=mesh,
        scratch_types=[
            pltpu.VMEM((n_ch, _IDX_CHUNK), jnp.int32),
            pltpu.VMEM((b_per_w, W), jnp.float32),
            pltpu.SemaphoreType.DMA,
        ],
    )
    def gather(idx_hbm, table_hbm, out_hbm, idx_v, rows_v, sem):
        wid = lax.axis_index("s") * info.num_cores + lax.axis_index("c")
        base = wid * b_per_w
        pltpu.sync_copy(idx_hbm.at[wid], idx_v)
        copies = [
            pltpu.async_copy(
                table_hbm.at[idx_v.at[j]],
                rows_v.at[pl.ds(j * _IDX_CHUNK, _IDX_CHUNK)],
                sem,
            )
            for j in range(n_ch)
        ]
        for c in copies:
            c.wait()
        pltpu.sync_copy(rows_v, out_hbm.at[pl.ds(base, b_per_w)])

    return gather


def _sample_body(pairs_ref, gg_ref, act_ref, probs_ref):
    pairs = jnp.maximum(pairs_ref[...], 1e-30)
    gg = gg_ref[...]
    A = pairs.shape[1] // 2
    z = jnp.log(pairs) + gg
    m = jnp.max(z, axis=1, keepdims=True)
    ii = lax.broadcasted_iota(jnp.int32, z.shape, 1)
    act_ref[...] = jnp.min(jnp.where(z == m, ii, z.shape[1]), axis=1) & (A - 1)
    sel = gg[:, :A] > gg[:, A:]
    probs_ref[...] = jnp.where(sel, pairs[:, :A], pairs[:, A:])


@functools.cache
def _sample_fn(B, W, blk):
    A = W // 2
    return pl.pallas_call(
        _sample_body,
        grid=(B // blk,),
        in_specs=[
            pl.BlockSpec((blk, W), lambda i: (i, 0)),
            pl.BlockSpec((blk, W), lambda i: (i, 0)),
        ],
        out_specs=[
            pl.BlockSpec((blk,), lambda i: (i,)),
            pl.BlockSpec((blk, A), lambda i: (i, 0)),
        ],
        out_shape=[
            jax.ShapeDtypeStruct((B,), jnp.int32),
            jax.ShapeDtypeStruct((B, A), jnp.float32),
        ],
    )


def kernel(state, model):
    B = state.shape[0]
    V, A = model.shape
    info = plsc.get_sparse_core_info()
    nw = info.num_cores * info.num_subcores
    b_per_w = B // nw
    half = _TBLK // 2
    k = (state // _TBLK) * half + (state & (half - 1))
    h = ((state // half) & 1)[:, None].astype(jnp.bool_)
    g = jax.random.gumbel(jax.random.key(42), (B, A), jnp.float32)
    gg = jnp.concatenate(
        [jnp.where(h, -jnp.inf, g), jnp.where(h, g, -jnp.inf)], axis=1
    )
    table2 = _pairize_fn(A, V)(model.T)
    idx = k.reshape(nw, b_per_w // _IDX_CHUNK, _IDX_CHUNK)
    pairs = _gather_fn(B, table2.shape[0], 2 * A)(idx, table2)
    actions, action_probs = _sample_fn(B, 2 * A, 2048)(pairs, gg)
    return actions, action_probs
